# baseline (speedup 1.0000x reference)
import functools

import jax
import jax.numpy as jnp
from jax import lax
from jax.experimental import pallas as pl
from jax.experimental.pallas import tpu as pltpu
from jax.experimental.pallas import tpu_sc as plsc

NUM_ROWS = 16384
NUM_FEATS = 26
DIM = 32

_NC = 2
_NS = 16
_NW = _NC * _NS
_L = 16

_RW = NUM_ROWS // _NW
_IBLK = _RW // 128
_KBLK = DIM // 8


def _body(idx_hbm, tbl_hbm, out_hbm,
          idxj0, idxj1, buf0, buf1, tbuf0, tbuf1,
          semg0, semg1, semo0, semo1):
    w = lax.axis_index("s") * _NC + lax.axis_index("c")
    i0 = w * _RW
    idxjs = (idxj0, idxj1)
    bufs = (buf0, buf1)
    tbufs = (tbuf0, tbuf1)
    semgs = (semg0, semg1)
    semos = (semo0, semo1)
    iota = lax.iota(jnp.int32, _L)

    def prep(j, s):
        pltpu.sync_copy(idx_hbm.at[pl.ds(j * NUM_ROWS + i0, _RW)], idxjs[s])
        pltpu.async_copy(tbl_hbm.at[idxjs[s]], bufs[s], semgs[s])

    def wait_gather(s):
        pltpu.make_async_copy(tbl_hbm.at[idxjs[s]], bufs[s], semgs[s]).wait()

    def transpose(s):
        buf = bufs[s]
        tbuf = tbufs[s]

        @plsc.parallel_loop(0, _KBLK * _IBLK, unroll=2)
        def _(t):
            kblk = t // _IBLK
            ibl = t % _IBLK
            kbase = kblk * 8
            pbase = iota + ibl * 128
            kvecs = [jnp.zeros((_L,), jnp.int32) + (kbase + kk) for kk in range(8)]
            for ii0 in range(0, 128, 2 * _L):
                pvecs = [pbase + ii0, pbase + (ii0 + _L)]
                tvs = [
                    plsc.load_gather(buf, [pvecs[h], kvecs[kk]])
                    for h in range(2)
                    for kk in range(8)
                ]
                for h in range(2):
                    for kk in range(8):
                        tbuf[kblk, ibl, kk, pl.ds(ii0 + h * _L, _L)] = tvs[h * 8 + kk]

    def start_out(s, j):
        pltpu.async_copy(
            tbufs[s], out_hbm.at[j, :, pl.ds(_IBLK * w, _IBLK)], semos[s]
        )

    def wait_out(s, j):
        pltpu.make_async_copy(
            tbufs[s], out_hbm.at[j, :, pl.ds(_IBLK * w, _IBLK)], semos[s]
        ).wait()

    prep(0, 0)
    prep(1, 1)

    @pl.loop(0, NUM_FEATS, step=2)
    def _(j0):
        for s in range(2):
            j = j0 + s
            wait_gather(s)

            @pl.when(j >= 2)
            def _():
                wait_out(s, j)

            transpose(s)
            start_out(s, j)

            @pl.when(j + 2 < NUM_FEATS)
            def _():
                prep(j + 2, s)

    wait_out(0, NUM_FEATS - 2)
    wait_out(1, NUM_FEATS - 1)


@functools.partial(
    pl.kernel,
    out_type=jax.ShapeDtypeStruct((NUM_FEATS, _KBLK, 128, 8, 128), jnp.float32),
    mesh=plsc.VectorSubcoreMesh(core_axis_name="c", subcore_axis_name="s"),
    scratch_types=[
        pltpu.VMEM((_RW,), jnp.int32),
        pltpu.VMEM((_RW,), jnp.int32),
        pltpu.VMEM((_RW, DIM), jnp.float32),
        pltpu.VMEM((_RW, DIM), jnp.float32),
        pltpu.VMEM((_KBLK, _IBLK, 8, 128), jnp.float32),
        pltpu.VMEM((_KBLK, _IBLK, 8, 128), jnp.float32),
        pltpu.SemaphoreType.DMA,
        pltpu.SemaphoreType.DMA,
        pltpu.SemaphoreType.DMA,
        pltpu.SemaphoreType.DMA,
    ],
    compiler_params=pltpu.CompilerParams(
        use_tc_tiling_on_sc=False, needs_layout_passes=False
    ),
)
def _gather(idx_hbm, tbl_hbm, out_hbm, *scratch):
    _body(idx_hbm, tbl_hbm, out_hbm, *scratch)


def kernel(indices, table):
    idx_fm = indices.T.reshape(NUM_ROWS * NUM_FEATS).astype(jnp.int32)
    tbl_lin = lax.optimization_barrier(table.reshape(250000, 4 * DIM))
    tbl_lin = tbl_lin.reshape(1000000, DIM)
    out5 = _gather(idx_fm, tbl_lin)
    return out5.transpose(2, 4, 0, 1, 3).reshape(NUM_ROWS, NUM_FEATS, DIM)


# speedup vs baseline: 1.0171x; 1.0171x over previous
import functools

import jax
import jax.numpy as jnp
from jax import lax
from jax.experimental import pallas as pl
from jax.experimental.pallas import tpu as pltpu
from jax.experimental.pallas import tpu_sc as plsc

NUM_ROWS = 16384
NUM_FEATS = 26
DIM = 32

_NC = 2
_NS = 16
_NW = _NC * _NS
_L = 16

_RW = NUM_ROWS // _NW
_IBLK = _RW // 128
_KBLK = DIM // 8


def _body(idx_hbm, tbl_hbm, out_hbm,
          idxj0, idxj1, buf0, buf1, tbuf0, tbuf1,
          semg0, semg1, semo0, semo1):
    w = lax.axis_index("s") * _NC + lax.axis_index("c")
    i0 = w * _RW
    idxjs = (idxj0, idxj1)
    bufs = (buf0, buf1)
    tbufs = (tbuf0, tbuf1)
    semgs = (semg0, semg1)
    semos = (semo0, semo1)
    iota = lax.iota(jnp.int32, _L)

    def prep(j, s):
        pltpu.sync_copy(idx_hbm.at[pl.ds(j * NUM_ROWS + i0, _RW)], idxjs[s])
        pltpu.async_copy(tbl_hbm.at[idxjs[s]], bufs[s], semgs[s])

    def wait_gather(s):
        pltpu.make_async_copy(tbl_hbm.at[idxjs[s]], bufs[s], semgs[s]).wait()

    def transpose(s):
        buf = bufs[s]
        tbuf = tbufs[s]

        @plsc.parallel_loop(0, _KBLK * _IBLK, unroll=2)
        def _(t):
            kblk = t // _IBLK
            ibl = t % _IBLK
            kbase = kblk * 8
            pbase = iota + ibl * 128
            kvecs = [jnp.zeros((_L,), jnp.int32) + (kbase + kk) for kk in range(8)]
            for ii0 in range(0, 128, _L):
                pvec = pbase + ii0
                tvs = [plsc.load_gather(buf, [pvec, kvecs[kk]]) for kk in range(8)]
                for kk in range(8):
                    tbuf[kblk, ibl, kk, pl.ds(ii0, _L)] = tvs[kk]

    def start_out(s, j):
        pltpu.async_copy(
            tbufs[s], out_hbm.at[j, :, pl.ds(_IBLK * w, _IBLK)], semos[s]
        )

    def wait_out(s, j):
        pltpu.make_async_copy(
            tbufs[s], out_hbm.at[j, :, pl.ds(_IBLK * w, _IBLK)], semos[s]
        ).wait()

    prep(0, 0)
    prep(1, 1)

    @pl.loop(0, NUM_FEATS, step=2)
    def _(j0):
        for s in range(2):
            j = j0 + s
            wait_gather(s)

            @pl.when(j >= 2)
            def _():
                wait_out(s, j)

            transpose(s)
            start_out(s, j)

            @pl.when(j + 2 < NUM_FEATS)
            def _():
                prep(j + 2, s)

    wait_out(0, NUM_FEATS - 2)
    wait_out(1, NUM_FEATS - 1)


@functools.partial(
    pl.kernel,
    out_type=jax.ShapeDtypeStruct((NUM_FEATS, _KBLK, 128, 8, 128), jnp.float32),
    mesh=plsc.VectorSubcoreMesh(core_axis_name="c", subcore_axis_name="s"),
    scratch_types=[
        pltpu.VMEM((_RW,), jnp.int32),
        pltpu.VMEM((_RW,), jnp.int32),
        pltpu.VMEM((_RW, DIM), jnp.float32),
        pltpu.VMEM((_RW, DIM), jnp.float32),
        pltpu.VMEM((_KBLK, _IBLK, 8, 128), jnp.float32),
        pltpu.VMEM((_KBLK, _IBLK, 8, 128), jnp.float32),
        pltpu.SemaphoreType.DMA,
        pltpu.SemaphoreType.DMA,
        pltpu.SemaphoreType.DMA,
        pltpu.SemaphoreType.DMA,
    ],
    compiler_params=pltpu.CompilerParams(
        use_tc_tiling_on_sc=False, needs_layout_passes=False
    ),
)
def _gather(idx_hbm, tbl_hbm, out_hbm, *scratch):
    _body(idx_hbm, tbl_hbm, out_hbm, *scratch)


def kernel(indices, table):
    idx_fm = indices.T.reshape(NUM_ROWS * NUM_FEATS).astype(jnp.int32)
    tbl_lin = lax.optimization_barrier(table.reshape(250000, 4 * DIM))
    tbl_lin = tbl_lin.reshape(1000000, DIM)
    out5 = _gather(idx_fm, tbl_lin)
    return out5.transpose(2, 4, 0, 1, 3).reshape(NUM_ROWS, NUM_FEATS, DIM)
